# baseline (speedup 1.0000x reference)
import jax
import jax.numpy as jnp
from jax import lax
from jax.experimental import pallas as pl
from jax.experimental.pallas import tpu as pltpu
from jax.experimental.pallas import tpu_sc as plsc

VOCAB = 100000
D = 2048
B = 4
T = 2048
N = B * T

NC = 2
NS = 16
NW = NC * NS
ROWS_PER_W = N // NW
R = 16
NCHUNK = ROWS_PER_W // R
LANES = 16
DV = D // LANES


def _body(idx_hbm, tok_hbm, pos_hbm, out_hbm, idx_v, tok_v, pos_v, sem):
    c = lax.axis_index("c")
    s = lax.axis_index("s")
    wid = s * NC + c
    base = wid * ROWS_PER_W
    t0 = lax.rem(base, T)

    pltpu.sync_copy(idx_hbm.at[pl.ds(base, ROWS_PER_W)], idx_v)

    @pl.loop(0, NCHUNK)
    def chunk_loop(ci):
        r0 = ci * R
        gather = pltpu.async_copy(tok_hbm.at[idx_v.at[pl.ds(r0, R)]], tok_v, sem)
        pltpu.sync_copy(pos_hbm.at[pl.ds(t0 + r0, R)], pos_v)
        gather.wait()

        @pl.loop(0, R)
        def row_loop(r):
            @pl.loop(0, DV)
            def col_loop(j):
                sl = pl.ds(j * LANES, LANES)
                pos_v[r, sl] = pos_v[r, sl] + tok_v[r, sl]

        pltpu.sync_copy(pos_v, out_hbm.at[pl.ds(base + r0, R)])


@jax.jit
def _run(idx_flat, tok_table, pos_table):
    mesh = plsc.VectorSubcoreMesh(
        core_axis_name="c", subcore_axis_name="s", num_cores=NC, num_subcores=NS
    )
    f = pl.kernel(
        _body,
        out_type=jax.ShapeDtypeStruct((N, D), jnp.float32),
        mesh=mesh,
        scratch_types=[
            pltpu.VMEM((ROWS_PER_W,), jnp.int32),
            pltpu.VMEM((R, D), jnp.float32),
            pltpu.VMEM((R, D), jnp.float32),
            pltpu.SemaphoreType.DMA,
        ],
    )
    return f(idx_flat, tok_table, pos_table)


def kernel(idx, tok_table, pos_table):
    idx_flat = idx.reshape(N).astype(jnp.int32)
    out = _run(idx_flat, tok_table, pos_table)
    return out.reshape(B, T, D)


# speedup vs baseline: 2.5928x; 2.5928x over previous
import jax
import jax.numpy as jnp
from jax import lax
from jax.experimental import pallas as pl
from jax.experimental.pallas import tpu as pltpu
from jax.experimental.pallas import tpu_sc as plsc

VOCAB = 100000
D = 2048
B = 4
T = 2048
N = B * T

NC = 2
NS = 16
NW = NC * NS
ROWS_PER_W = N // NW
R = 8
NCHUNK = ROWS_PER_W // R
LANES = 16
DV = D // LANES


def _add_chunk(tok_v, pos_v):
    @pl.loop(0, R)
    def row_loop(r):
        @pl.loop(0, DV, unroll=16)
        def col_loop(j):
            sl = pl.ds(j * LANES, LANES)
            plsc.addupdate(pos_v.at[r, sl], tok_v[r, sl])


def _body(
    idx_hbm, tok_hbm, pos_hbm, out_hbm,
    idx_v, tok0, tok1, pos0, pos1, sg0, sg1, sp0, sp1, sw0, sw1,
):
    c = lax.axis_index("c")
    s = lax.axis_index("s")
    wid = s * NC + c
    base = wid * ROWS_PER_W
    t0 = lax.rem(base, T)

    toks = (tok0, tok1)
    poss = (pos0, pos1)
    sgs = (sg0, sg1)
    sps = (sp0, sp1)
    sws = (sw0, sw1)

    pltpu.sync_copy(idx_hbm.at[pl.ds(base, ROWS_PER_W)], idx_v)

    def fetch(cur, b):
        r0 = cur * R
        pltpu.async_copy(tok_hbm.at[idx_v.at[pl.ds(r0, R)]], toks[b], sgs[b])
        pltpu.async_copy(pos_hbm.at[pl.ds(t0 + r0, R)], poss[b], sps[b])

    def wait_fetch(b):
        pltpu.make_async_copy(tok_hbm.at[pl.ds(0, R)], toks[b], sgs[b]).wait()
        pltpu.make_async_copy(pos_hbm.at[pl.ds(0, R)], poss[b], sps[b]).wait()

    def wait_write(cur, b):
        pltpu.make_async_copy(
            poss[b], out_hbm.at[pl.ds(base + cur * R, R)], sws[b]
        ).wait()

    fetch(0, 0)

    @pl.loop(0, NCHUNK, step=2)
    def chunk_loop(ci):
        for b in range(2):
            cur = ci + b

            @pl.when(cur + 1 < NCHUNK)
            def _():
                @pl.when(cur >= 1)
                def _():
                    wait_write(cur - 1, 1 - b)

                fetch(cur + 1, 1 - b)

            wait_fetch(b)
            _add_chunk(toks[b], poss[b])
            pltpu.async_copy(
                poss[b], out_hbm.at[pl.ds(base + cur * R, R)], sws[b]
            )

    wait_write(NCHUNK - 2, NCHUNK % 2)
    wait_write(NCHUNK - 1, (NCHUNK - 1) % 2)


@jax.jit
def _run(idx_flat, tok_table, pos_table):
    mesh = plsc.VectorSubcoreMesh(
        core_axis_name="c", subcore_axis_name="s", num_cores=NC, num_subcores=NS
    )
    f = pl.kernel(
        _body,
        out_type=jax.ShapeDtypeStruct((N, D), jnp.float32),
        mesh=mesh,
        scratch_types=[
            pltpu.VMEM((ROWS_PER_W,), jnp.int32),
            pltpu.VMEM((R, D), jnp.float32),
            pltpu.VMEM((R, D), jnp.float32),
            pltpu.VMEM((R, D), jnp.float32),
            pltpu.VMEM((R, D), jnp.float32),
            pltpu.SemaphoreType.DMA,
            pltpu.SemaphoreType.DMA,
            pltpu.SemaphoreType.DMA,
            pltpu.SemaphoreType.DMA,
            pltpu.SemaphoreType.DMA,
            pltpu.SemaphoreType.DMA,
        ],
    )
    return f(idx_flat, tok_table, pos_table)


def kernel(idx, tok_table, pos_table):
    idx_flat = idx.reshape(N).astype(jnp.int32)
    out = _run(idx_flat, tok_table, pos_table)
    return out.reshape(B, T, D)


# speedup vs baseline: 2.8076x; 1.0828x over previous
import jax
import jax.numpy as jnp
from jax import lax
from jax.experimental import pallas as pl
from jax.experimental.pallas import tpu as pltpu
from jax.experimental.pallas import tpu_sc as plsc

VOCAB = 100000
D = 2048
B = 4
T = 2048
N = B * T

NC = 2
NS = 16
NW = NC * NS
T_PER_W = T // NW
R = 8
NCHUNK = T_PER_W // R
NSTEP = NCHUNK * B
LANES = 16
DV = D // LANES


def _add_chunk(tok_v, pos_v):
    @pl.loop(0, R)
    def row_loop(r):
        @pl.loop(0, DV, unroll=16)
        def col_loop(j):
            sl = pl.ds(j * LANES, LANES)
            plsc.addupdate(tok_v.at[r, sl], pos_v[r, sl])


def _body(
    idx_hbm, tok_hbm, pos_hbm, out_hbm,
    idx_v, tok0, tok1, pos0, pos1, sg0, sg1, sp0, sp1, sw0, sw1,
):
    c = lax.axis_index("c")
    s = lax.axis_index("s")
    wid = s * NC + c
    t0 = wid * T_PER_W

    toks = (tok0, tok1)
    poss = (pos0, pos1)
    sgs = (sg0, sg1)
    sps = (sp0, sp1)
    sws = (sw0, sw1)

    for b in range(B):
        pltpu.sync_copy(
            idx_hbm.at[pl.ds(b * T + t0, T_PER_W)],
            idx_v.at[pl.ds(b * T_PER_W, T_PER_W)],
        )

    def fetch_tok(ci, b, slot):
        off = b * T_PER_W + ci * R
        pltpu.async_copy(tok_hbm.at[idx_v.at[pl.ds(off, R)]], toks[slot], sgs[slot])

    def fetch_pos(ci, pslot):
        pltpu.async_copy(pos_hbm.at[pl.ds(t0 + ci * R, R)], poss[pslot], sps[pslot])

    def out_rows(ci, b):
        return out_hbm.at[pl.ds(b * T + t0 + ci * R, R)]

    def wait_tok(slot):
        pltpu.make_async_copy(tok_hbm.at[pl.ds(0, R)], toks[slot], sgs[slot]).wait()

    def wait_pos(pslot):
        pltpu.make_async_copy(pos_hbm.at[pl.ds(0, R)], poss[pslot], sps[pslot]).wait()

    def wait_write(slot):
        pltpu.make_async_copy(toks[slot], out_hbm.at[pl.ds(0, R)], sws[slot]).wait()

    fetch_tok(0, 0, 0)
    fetch_pos(0, 0)

    @pl.loop(0, NCHUNK, step=2)
    def chunk_loop(cio):
        for cc in range(2):
            ci = cio + cc
            for b in range(B):
                k = ci * B + b
                slot = b % 2

                nslot = (b + 1) % 2
                if b < B - 1:
                    @pl.when(k >= 1)
                    def _():
                        wait_write(nslot)

                    fetch_tok(ci, b + 1, nslot)
                else:
                    @pl.when(ci < NCHUNK - 1)
                    def _():
                        wait_write(nslot)
                        fetch_tok(ci + 1, 0, nslot)

                if b == 0:
                    wait_pos(cc % 2)

                    @pl.when(ci < NCHUNK - 1)
                    def _():
                        fetch_pos(ci + 1, (cc + 1) % 2)

                wait_tok(slot)
                _add_chunk(toks[slot], poss[cc % 2])
                pltpu.async_copy(toks[slot], out_rows(ci, b), sws[slot])

    wait_write((NSTEP - 2) % 2)
    wait_write((NSTEP - 1) % 2)


@jax.jit
def _run(idx_flat, tok_table, pos_table):
    mesh = plsc.VectorSubcoreMesh(
        core_axis_name="c", subcore_axis_name="s", num_cores=NC, num_subcores=NS
    )
    f = pl.kernel(
        _body,
        out_type=jax.ShapeDtypeStruct((N, D), jnp.float32),
        mesh=mesh,
        scratch_types=[
            pltpu.VMEM((B * T_PER_W,), jnp.int32),
            pltpu.VMEM((R, D), jnp.float32),
            pltpu.VMEM((R, D), jnp.float32),
            pltpu.VMEM((R, D), jnp.float32),
            pltpu.VMEM((R, D), jnp.float32),
            pltpu.SemaphoreType.DMA,
            pltpu.SemaphoreType.DMA,
            pltpu.SemaphoreType.DMA,
            pltpu.SemaphoreType.DMA,
            pltpu.SemaphoreType.DMA,
            pltpu.SemaphoreType.DMA,
        ],
    )
    return f(idx_flat, tok_table, pos_table)


def kernel(idx, tok_table, pos_table):
    idx_flat = idx.reshape(N).astype(jnp.int32)
    out = _run(idx_flat, tok_table, pos_table)
    return out.reshape(B, T, D)


# speedup vs baseline: 3.0334x; 1.0804x over previous
import jax
import jax.numpy as jnp
from jax import lax
from jax.experimental import pallas as pl
from jax.experimental.pallas import tpu as pltpu
from jax.experimental.pallas import tpu_sc as plsc

VOCAB = 100000
D = 2048
B = 4
T = 2048
N = B * T

NC = 2
NS = 16
NW = NC * NS
T_PER_W = T // NW
R = 8
NCHUNK = T_PER_W // R
NSTEP = NCHUNK * B
NSLOT = 4
LANES = 16
DV = D // LANES


def _add_chunk(tok_v, pos_v):
    @pl.loop(0, R)
    def row_loop(r):
        @pl.loop(0, DV, unroll=16)
        def col_loop(j):
            sl = pl.ds(j * LANES, LANES)
            plsc.addupdate(tok_v.at[r, sl], pos_v[r, sl])


def _body(
    idx_hbm, tok_hbm, pos_hbm, out_hbm,
    idx_v, tok0, tok1, tok2, tok3, pos0, pos1,
    sg0, sg1, sg2, sg3, sw0, sw1, sw2, sw3, sp0, sp1, si,
):
    c = lax.axis_index("c")
    s = lax.axis_index("s")
    wid = s * NC + c
    t0 = wid * T_PER_W

    toks = (tok0, tok1, tok2, tok3)
    poss = (pos0, pos1)
    sgs = (sg0, sg1, sg2, sg3)
    sws = (sw0, sw1, sw2, sw3)
    sps = (sp0, sp1)

    for b in range(B):
        pltpu.async_copy(
            idx_hbm.at[pl.ds(b * T + t0, T_PER_W)],
            idx_v.at[pl.ds(b * T_PER_W, T_PER_W)],
            si,
        )
    for b in range(B):
        pltpu.make_async_copy(
            idx_hbm.at[pl.ds(0, T_PER_W)],
            idx_v.at[pl.ds(b * T_PER_W, T_PER_W)],
            si,
        ).wait()

    def fetch_tok(k):
        ci, b = k // B, k % B
        slot = k % NSLOT
        off = b * T_PER_W + ci * R
        pltpu.async_copy(tok_hbm.at[idx_v.at[pl.ds(off, R)]], toks[slot], sgs[slot])

    def wait_tok(k):
        slot = k % NSLOT
        pltpu.make_async_copy(tok_hbm.at[pl.ds(0, R)], toks[slot], sgs[slot]).wait()

    def start_write(k):
        ci, b = k // B, k % B
        slot = k % NSLOT
        pltpu.async_copy(
            toks[slot], out_hbm.at[pl.ds(b * T + t0 + ci * R, R)], sws[slot]
        )

    def wait_write(k):
        slot = k % NSLOT
        pltpu.make_async_copy(toks[slot], out_hbm.at[pl.ds(0, R)], sws[slot]).wait()

    def fetch_pos(ci):
        pltpu.async_copy(
            pos_hbm.at[pl.ds(t0 + ci * R, R)], poss[ci % 2], sps[ci % 2]
        )

    def wait_pos(ci):
        pltpu.make_async_copy(
            pos_hbm.at[pl.ds(0, R)], poss[ci % 2], sps[ci % 2]
        ).wait()

    fetch_pos(0)
    fetch_tok(0)
    fetch_tok(1)

    for k in range(NSTEP):
        ci, b = k // B, k % B
        if k + 2 < NSTEP:
            if k >= 2:
                wait_write(k - 2)
            fetch_tok(k + 2)
        if b == 0:
            wait_pos(ci)
            if ci + 1 < NCHUNK:
                fetch_pos(ci + 1)
        wait_tok(k)
        _add_chunk(toks[k % NSLOT], poss[ci % 2])
        start_write(k)

    for k in range(NSTEP - 4, NSTEP):
        wait_write(k)


@jax.jit
def _run(idx_flat, tok_table, pos_table):
    mesh = plsc.VectorSubcoreMesh(
        core_axis_name="c", subcore_axis_name="s", num_cores=NC, num_subcores=NS
    )
    f = pl.kernel(
        _body,
        out_type=jax.ShapeDtypeStruct((N, D), jnp.float32),
        mesh=mesh,
        scratch_types=[
            pltpu.VMEM((B * T_PER_W,), jnp.int32),
            pltpu.VMEM((R, D), jnp.float32),
            pltpu.VMEM((R, D), jnp.float32),
            pltpu.VMEM((R, D), jnp.float32),
            pltpu.VMEM((R, D), jnp.float32),
            pltpu.VMEM((R, D), jnp.float32),
            pltpu.VMEM((R, D), jnp.float32),
            pltpu.SemaphoreType.DMA,
            pltpu.SemaphoreType.DMA,
            pltpu.SemaphoreType.DMA,
            pltpu.SemaphoreType.DMA,
            pltpu.SemaphoreType.DMA,
            pltpu.SemaphoreType.DMA,
            pltpu.SemaphoreType.DMA,
            pltpu.SemaphoreType.DMA,
            pltpu.SemaphoreType.DMA,
            pltpu.SemaphoreType.DMA,
            pltpu.SemaphoreType.DMA,
        ],
    )
    return f(idx_flat, tok_table, pos_table)


def kernel(idx, tok_table, pos_table):
    idx_flat = idx.reshape(N).astype(jnp.int32)
    out = _run(idx_flat, tok_table, pos_table)
    return out.reshape(B, T, D)


# speedup vs baseline: 3.0500x; 1.0055x over previous
import jax
import jax.numpy as jnp
from jax import lax
from jax.experimental import pallas as pl
from jax.experimental.pallas import tpu as pltpu
from jax.experimental.pallas import tpu_sc as plsc

VOCAB = 100000
D = 2048
B = 4
T = 2048
N = B * T

NC = 2
NS = 16
NW = NC * NS
T_PER_W = T // NW
R = 8
NCHUNK = T_PER_W // R
NSTEP = NCHUNK * B
NSLOT = 5
LANES = 16
DV = D // LANES


def _add_chunk(tok_v, pos_v):
    @pl.loop(0, R)
    def row_loop(r):
        @pl.loop(0, DV, unroll=16)
        def col_loop(j):
            sl = pl.ds(j * LANES, LANES)
            plsc.addupdate(tok_v.at[r, sl], pos_v[r, sl])


def _body(
    idx_hbm, tok_hbm, pos_hbm, out_hbm,
    idx_v, tok0, tok1, tok2, tok3, tok4, pos0, pos1,
    sg0, sg1, sg2, sg3, sg4, sw0, sw1, sw2, sw3, sw4, sp0, sp1, si,
):
    c = lax.axis_index("c")
    s = lax.axis_index("s")
    wid = s * NC + c
    t0 = wid * T_PER_W

    toks = (tok0, tok1, tok2, tok3, tok4)
    poss = (pos0, pos1)
    sgs = (sg0, sg1, sg2, sg3, sg4)
    sws = (sw0, sw1, sw2, sw3, sw4)
    sps = (sp0, sp1)

    for b in range(B):
        pltpu.async_copy(
            idx_hbm.at[pl.ds(b * T + t0, T_PER_W)],
            idx_v.at[pl.ds(b * T_PER_W, T_PER_W)],
            si,
        )
    for b in range(B):
        pltpu.make_async_copy(
            idx_hbm.at[pl.ds(0, T_PER_W)],
            idx_v.at[pl.ds(b * T_PER_W, T_PER_W)],
            si,
        ).wait()

    def fetch_tok(k):
        ci, b = k // B, k % B
        slot = k % NSLOT
        off = b * T_PER_W + ci * R
        pltpu.async_copy(tok_hbm.at[idx_v.at[pl.ds(off, R)]], toks[slot], sgs[slot])

    def wait_tok(k):
        slot = k % NSLOT
        pltpu.make_async_copy(tok_hbm.at[pl.ds(0, R)], toks[slot], sgs[slot]).wait()

    def start_write(k):
        ci, b = k // B, k % B
        slot = k % NSLOT
        pltpu.async_copy(
            toks[slot], out_hbm.at[pl.ds(b * T + t0 + ci * R, R)], sws[slot]
        )

    def wait_write(k):
        slot = k % NSLOT
        pltpu.make_async_copy(toks[slot], out_hbm.at[pl.ds(0, R)], sws[slot]).wait()

    def fetch_pos(ci):
        pltpu.async_copy(
            pos_hbm.at[pl.ds(t0 + ci * R, R)], poss[ci % 2], sps[ci % 2]
        )

    def wait_pos(ci):
        pltpu.make_async_copy(
            pos_hbm.at[pl.ds(0, R)], poss[ci % 2], sps[ci % 2]
        ).wait()

    fetch_pos(0)
    fetch_tok(0)
    fetch_tok(1)
    fetch_tok(2)

    for k in range(NSTEP):
        ci, b = k // B, k % B
        if k + 3 < NSTEP:
            if k >= 2:
                wait_write(k - 2)
            fetch_tok(k + 3)
        if b == 0:
            wait_pos(ci)
            if ci + 1 < NCHUNK:
                fetch_pos(ci + 1)
        wait_tok(k)
        _add_chunk(toks[k % NSLOT], poss[ci % 2])
        start_write(k)

    for k in range(NSTEP - 5, NSTEP):
        wait_write(k)


@jax.jit
def _run(idx_flat, tok_table, pos_table):
    mesh = plsc.VectorSubcoreMesh(
        core_axis_name="c", subcore_axis_name="s", num_cores=NC, num_subcores=NS
    )
    f = pl.kernel(
        _body,
        out_type=jax.ShapeDtypeStruct((N, D), jnp.float32),
        mesh=mesh,
        scratch_types=[
            pltpu.VMEM((B * T_PER_W,), jnp.int32),
            pltpu.VMEM((R, D), jnp.float32),
            pltpu.VMEM((R, D), jnp.float32),
            pltpu.VMEM((R, D), jnp.float32),
            pltpu.VMEM((R, D), jnp.float32),
            pltpu.VMEM((R, D), jnp.float32),
            pltpu.VMEM((R, D), jnp.float32),
            pltpu.VMEM((R, D), jnp.float32),
            pltpu.SemaphoreType.DMA,
            pltpu.SemaphoreType.DMA,
            pltpu.SemaphoreType.DMA,
            pltpu.SemaphoreType.DMA,
            pltpu.SemaphoreType.DMA,
            pltpu.SemaphoreType.DMA,
            pltpu.SemaphoreType.DMA,
            pltpu.SemaphoreType.DMA,
            pltpu.SemaphoreType.DMA,
            pltpu.SemaphoreType.DMA,
            pltpu.SemaphoreType.DMA,
            pltpu.SemaphoreType.DMA,
            pltpu.SemaphoreType.DMA,
        ],
    )
    return f(idx_flat, tok_table, pos_table)


def kernel(idx, tok_table, pos_table):
    idx_flat = idx.reshape(N).astype(jnp.int32)
    out = _run(idx_flat, tok_table, pos_table)
    return out.reshape(B, T, D)
